# baseline (speedup 1.0000x reference)
import functools
import math

import jax
import jax.numpy as jnp
import numpy as np
from jax import lax
from jax.experimental import pallas as pl
from jax.experimental.pallas import tpu as pltpu
from jax.experimental.pallas import tpu_sc as plsc

_B = 2
_N = 20
_C = 256
_T = 4
_HWS = (64 * 64, 32 * 32, 16 * 16, 8 * 8)
_KS = tuple(int(math.floor(0.0625 * hw)) for hw in _HWS)
_KPADS = tuple(max(8, k) for k in _KS)
_KOFFS = (0, 256, 320, 336)
_KSUM = sum(_KS)
_NBT = _B * _N * _T

_MSB = np.int32(-2147483648)
_LOW31 = np.int32(2147483647)


def _monotone_key(x_f32):
    fi = lax.bitcast_convert_type(x_f32, jnp.int32)
    return jnp.where(fi >= 0, fi, fi ^ _LOW31)


def _row_cumsum(x, hw):
    n = x.shape[0]
    if hw <= 128:
        io_r = lax.broadcasted_iota(jnp.int32, (hw, hw), 0)
        io_c = lax.broadcasted_iota(jnp.int32, (hw, hw), 1)
        tri_incl = (io_r <= io_c).astype(jnp.float32)
        return jnp.dot(x, tri_incl, preferred_element_type=jnp.float32)
    r = hw // 128
    io_r = lax.broadcasted_iota(jnp.int32, (128, 128), 0)
    io_c = lax.broadcasted_iota(jnp.int32, (128, 128), 1)
    tri_incl = (io_r <= io_c).astype(jnp.float32)
    within = jnp.dot(x.reshape(n * r, 128), tri_incl,
                     preferred_element_type=jnp.float32).reshape(n, r, 128)
    blk_sum = jnp.sum(x.reshape(n, r, 128), axis=2)
    io_br = lax.broadcasted_iota(jnp.int32, (r, r), 0)
    io_bc = lax.broadcasted_iota(jnp.int32, (r, r), 1)
    tri_excl = (io_br < io_bc).astype(jnp.float32)
    base = jnp.dot(blk_sum, tri_excl, preferred_element_type=jnp.float32)
    return (within + base[:, :, None]).reshape(n, hw)


def _topk_idx(mask, hw, k):
    n = mask.shape[0]
    ikey = _monotone_key(mask)
    tpat = jnp.zeros((n, 1), jnp.int32)
    for bit in range(31, -1, -1):
        bitc = np.int32(np.uint32(1 << bit))
        trial = tpat | bitc
        s_trial = trial ^ _MSB
        cnt = jnp.sum((ikey >= s_trial).astype(jnp.float32), axis=1,
                      keepdims=True)
        tpat = jnp.where(cnt >= float(k), trial, tpat)
    s_thr = tpat ^ _MSB
    gt = ikey > s_thr
    eq = ikey == s_thr
    cnt_gt = jnp.sum(gt.astype(jnp.float32), axis=1, keepdims=True)
    m_tie = float(k) - cnt_gt
    eqcum = _row_cumsum(eq.astype(jnp.float32), hw)
    sel = jnp.logical_or(gt, jnp.logical_and(eq, eqcum <= m_tie))
    selcum = _row_cumsum(sel.astype(jnp.float32), hw)

    chunks = []
    nch = (k + 7) // 8
    for c0 in range(nch):
        jv = lax.broadcasted_iota(jnp.int32, (1, 8, 1), 1).astype(
            jnp.float32) + float(c0 * 8)
        cmp = (selcum[:, None, :] <= jv).astype(jnp.float32)
        chunks.append(jnp.sum(cmp, axis=2))
    pos = jnp.concatenate(chunks, axis=1)[:, :k]
    return pos.astype(jnp.int32)


def _layer_norm_q(q_ref, w_ref, b_ref):
    q = q_ref[...].reshape(_N, _C)
    mu = jnp.mean(q, axis=-1, keepdims=True)
    var = jnp.mean((q - mu) ** 2, axis=-1, keepdims=True)
    return (q - mu) / jnp.sqrt(var + 1e-5) * w_ref[...].reshape(1, _C) \
        + b_ref[...].reshape(1, _C)


def _frame_body(hw, k, k_pad, srct_ref, q_ref, w_ref, b_ref, idx_ref):
    i = pl.program_id(0)
    qn = _layer_norm_q(q_ref, w_ref, b_ref)
    srct = srct_ref[...]
    maskt = jnp.dot(srct, qn.T, preferred_element_type=jnp.float32)
    mask = maskt.T
    idx = _topk_idx(mask, hw, k) + i * hw
    if k_pad > k:
        idx = jnp.concatenate(
            [idx, jnp.zeros((_N, k_pad - k), jnp.int32)], axis=1)
    idx_ref[...] = idx.reshape(1, 1, _N, k_pad)


def _make_scale_call(hw, k, k_pad):
    body = functools.partial(_frame_body, hw, k, k_pad)
    return pl.pallas_call(
        body,
        grid=(_B * _T,),
        in_specs=[
            pl.BlockSpec((hw, _C), lambda i: (i, 0)),
            pl.BlockSpec((1, _N, _C), lambda i: (i // _T, 0, 0)),
            pl.BlockSpec((1, _C), lambda i: (0, 0)),
            pl.BlockSpec((1, _C), lambda i: (0, 0)),
        ],
        out_specs=[
            pl.BlockSpec((1, 1, _N, k_pad), lambda i: (i // _T, i % _T, 0, 0)),
        ],
        out_shape=[
            jax.ShapeDtypeStruct((_B, _T, _N, k_pad), jnp.int32),
        ],
    )


def _sc_gather(tables, idxs):
    mesh = plsc.VectorSubcoreMesh(core_axis_name="c", subcore_axis_name="s")
    rows_per_tile = _NBT // 32

    @functools.partial(
        pl.kernel,
        mesh=mesh,
        out_type=jax.ShapeDtypeStruct((_NBT, _KSUM, _C), jnp.float32),
        scratch_types=[
            pltpu.VMEM((128,), jnp.int32),
            pltpu.VMEM((128, _C), jnp.float32),
            pltpu.SemaphoreType.DMA,
        ],
    )
    def k(t0h, t1h, t2h, t3h, i0h, i1h, i2h, i3h, out, idx_v, rows_v, sem):
        wid = lax.axis_index("s") * 2 + lax.axis_index("c")
        tbls = (t0h, t1h, t2h, t3h)
        idxs_h = (i0h, i1h, i2h, i3h)

        def task(a, carry):
            r = wid * rows_per_tile + a
            b = r // (_N * _T)
            n = (r // _T) % _N
            t = r % _T
            bt = b * _T + t
            for s in range(4):
                kk, kp, off = _KS[s], _KPADS[s], _KOFFS[s]
                idx_off = (bt * _N + n) * kp
                for c0 in range(0, kp, 128):
                    cw = min(128, kp - c0)
                    wr = min(kk - c0, cw)
                    idx_sub = idx_v.at[pl.ds(0, cw)]
                    pltpu.sync_copy(idxs_h[s].at[pl.ds(idx_off + c0, cw)],
                                    idx_sub)
                    pltpu.async_copy(tbls[s].at[idx_sub],
                                     rows_v.at[pl.ds(0, cw)], sem).wait()
                    pltpu.sync_copy(rows_v.at[pl.ds(0, wr)],
                                    out.at[r, pl.ds(off + c0, wr), :])
            return carry

        lax.fori_loop(0, rows_per_tile, task, 0)

    return k(*tables, *idxs)


def kernel(src0, src1, src2, src3, scan_queries, ln_w, ln_b):
    srcs = (src0, src1, src2, src3)
    w2 = ln_w.reshape(1, _C)
    b2 = ln_b.reshape(1, _C)
    tables = []
    for s, src in enumerate(srcs):
        hw = _HWS[s]
        tables.append(jnp.transpose(src.reshape(_B, _C, _T, hw),
                                    (0, 2, 3, 1)).reshape(_B * _T * hw, _C))
    idxs = [None] * 4
    for s in (3, 2, 1, 0):
        hw = _HWS[s]
        (idx,) = _make_scale_call(hw, _KS[s], _KPADS[s])(
            tables[s], scan_queries, w2, b2)
        idxs[s] = idx.reshape(-1)
    out = _sc_gather(tables, idxs)
    return out.reshape(_B, _N, _T, _KSUM, _C)


# speedup vs baseline: 1.1018x; 1.1018x over previous
import functools
import math

import jax
import jax.numpy as jnp
import numpy as np
from jax import lax
from jax.experimental import pallas as pl
from jax.experimental.pallas import tpu as pltpu
from jax.experimental.pallas import tpu_sc as plsc

_B = 2
_N = 20
_C = 256
_T = 4
_HWS = (64 * 64, 32 * 32, 16 * 16, 8 * 8)
_KS = tuple(int(math.floor(0.0625 * hw)) for hw in _HWS)
_KPADS = tuple(max(8, k) for k in _KS)
_KOFFS = (0, 256, 320, 336)
_KSUM = sum(_KS)
_KPAD_SUM = 344
_NBT = _B * _N * _T

_MSB = np.int32(-2147483648)
_LOW31 = np.int32(2147483647)


def _monotone_key(x_f32):
    fi = lax.bitcast_convert_type(x_f32, jnp.int32)
    return jnp.where(fi >= 0, fi, fi ^ _LOW31)


def _row_cumsum(x, hw):
    n = x.shape[0]
    if hw <= 128:
        io_r = lax.broadcasted_iota(jnp.int32, (hw, hw), 0)
        io_c = lax.broadcasted_iota(jnp.int32, (hw, hw), 1)
        tri_incl = (io_r <= io_c).astype(jnp.float32)
        return jnp.dot(x, tri_incl, preferred_element_type=jnp.float32)
    r = hw // 128
    io_r = lax.broadcasted_iota(jnp.int32, (128, 128), 0)
    io_c = lax.broadcasted_iota(jnp.int32, (128, 128), 1)
    tri_incl = (io_r <= io_c).astype(jnp.float32)
    within = jnp.dot(x.reshape(n * r, 128), tri_incl,
                     preferred_element_type=jnp.float32).reshape(n, r, 128)
    blk_sum = jnp.sum(x.reshape(n, r, 128), axis=2)
    io_br = lax.broadcasted_iota(jnp.int32, (r, r), 0)
    io_bc = lax.broadcasted_iota(jnp.int32, (r, r), 1)
    tri_excl = (io_br < io_bc).astype(jnp.float32)
    base = jnp.dot(blk_sum, tri_excl, preferred_element_type=jnp.float32)
    return (within + base[:, :, None]).reshape(n, hw)


def _topk_idx(mask, hw, k):
    n = mask.shape[0]
    ikey = _monotone_key(mask)
    tpat = jnp.zeros((n, 1), jnp.int32)
    for bit in range(31, -1, -1):
        bitc = np.int32(np.uint32(1 << bit))
        trial = tpat | bitc
        s_trial = trial ^ _MSB
        cnt = jnp.sum((ikey >= s_trial).astype(jnp.float32), axis=1,
                      keepdims=True)
        tpat = jnp.where(cnt >= float(k), trial, tpat)
    s_thr = tpat ^ _MSB
    gt = ikey > s_thr
    eq = ikey == s_thr
    cnt_gt = jnp.sum(gt.astype(jnp.float32), axis=1, keepdims=True)
    m_tie = float(k) - cnt_gt
    eqcum = _row_cumsum(eq.astype(jnp.float32), hw)
    sel = jnp.logical_or(gt, jnp.logical_and(eq, eqcum <= m_tie))
    selcum = _row_cumsum(sel.astype(jnp.float32), hw)

    chunks = []
    nch = (k + 7) // 8
    for c0 in range(nch):
        jv = lax.broadcasted_iota(jnp.int32, (1, 8, 1), 1).astype(
            jnp.float32) + float(c0 * 8)
        cmp = (selcum[:, None, :] <= jv).astype(jnp.float32)
        chunks.append(jnp.sum(cmp, axis=2))
    pos = jnp.concatenate(chunks, axis=1)[:, :k]
    return pos.astype(jnp.int32)


def _layer_norm_q(q_ref, w_ref, b_ref):
    q = q_ref[...].reshape(_N, _C)
    mu = jnp.mean(q, axis=-1, keepdims=True)
    var = jnp.mean((q - mu) ** 2, axis=-1, keepdims=True)
    return (q - mu) / jnp.sqrt(var + 1e-5) * w_ref[...].reshape(1, _C) \
        + b_ref[...].reshape(1, _C)


def _frame_body(hw, k, k_pad, srct_ref, q_ref, w_ref, b_ref, idx_ref):
    i = pl.program_id(0)
    qn = _layer_norm_q(q_ref, w_ref, b_ref)
    srct = srct_ref[...]
    maskt = jnp.dot(srct, qn.T, preferred_element_type=jnp.float32)
    mask = maskt.T
    idx = _topk_idx(mask, hw, k) + i * hw
    if k_pad > k:
        idx = jnp.concatenate(
            [idx, jnp.zeros((_N, k_pad - k), jnp.int32)], axis=1)
    idx_ref[...] = idx.reshape(1, 1, _N, k_pad)


def _make_scale_call(hw, k, k_pad):
    body = functools.partial(_frame_body, hw, k, k_pad)
    return pl.pallas_call(
        body,
        grid=(_B * _T,),
        in_specs=[
            pl.BlockSpec((hw, _C), lambda i: (i, 0)),
            pl.BlockSpec((1, _N, _C), lambda i: (i // _T, 0, 0)),
            pl.BlockSpec((1, _C), lambda i: (0, 0)),
            pl.BlockSpec((1, _C), lambda i: (0, 0)),
        ],
        out_specs=[
            pl.BlockSpec((1, 1, _N, k_pad), lambda i: (i // _T, i % _T, 0, 0)),
        ],
        out_shape=[
            jax.ShapeDtypeStruct((_B, _T, _N, k_pad), jnp.int32),
        ],
    )


def _sc_gather(tables, idxs):
    mesh = plsc.VectorSubcoreMesh(core_axis_name="c", subcore_axis_name="s")
    rows_per_tile = _NBT // 32

    @functools.partial(
        pl.kernel,
        mesh=mesh,
        out_type=[
            jax.ShapeDtypeStruct((_NBT * _KPAD_SUM, 128), jnp.float32),
            jax.ShapeDtypeStruct((_NBT * _KPAD_SUM, 128), jnp.float32),
        ],
        scratch_types=[
            pltpu.VMEM((128,), jnp.int32),
            pltpu.VMEM((128, _C), jnp.float32),
            pltpu.SemaphoreType.DMA,
        ],
    )
    def k(t0h, t1h, t2h, t3h, i0h, i1h, i2h, i3h, outl, outr,
          idx_v, rows_v, sem):
        wid = lax.axis_index("s") * 2 + lax.axis_index("c")
        tbls = (t0h, t1h, t2h, t3h)
        idxs_h = (i0h, i1h, i2h, i3h)

        def task(a, carry):
            r = wid * rows_per_tile + a
            b = r // (_N * _T)
            n = (r // _T) % _N
            t = r % _T
            bt = b * _T + t
            for s in range(4):
                kk, kp, off = _KS[s], _KPADS[s], _KOFFS[s]
                idx_off = (bt * _N + n) * kp
                for c0 in range(0, kp, 128):
                    cw = min(128, kp - c0)
                    wr = min(kk - c0, cw)
                    idx_sub = idx_v.at[pl.ds(0, cw)]
                    pltpu.sync_copy(idxs_h[s].at[pl.ds(idx_off + c0, cw)],
                                    idx_sub)
                    pltpu.async_copy(tbls[s].at[idx_sub],
                                     rows_v.at[pl.ds(0, cw)], sem).wait()
                    qbase = r * _KPAD_SUM + off + c0
                    pltpu.sync_copy(rows_v.at[pl.ds(0, wr), pl.ds(0, 128)],
                                    outl.at[pl.ds(qbase, wr), :])
                    pltpu.sync_copy(rows_v.at[pl.ds(0, wr), pl.ds(128, 128)],
                                    outr.at[pl.ds(qbase, wr), :])
            return carry

        lax.fori_loop(0, rows_per_tile, task, 0)

    return k(*tables, *idxs)


def _assemble_body(l_ref, r_ref, out_ref):
    lv = l_ref[...].reshape(_KPAD_SUM, 128)[:_KSUM]
    rv = r_ref[...].reshape(_KPAD_SUM, 128)[:_KSUM]
    out_ref[...] = jnp.concatenate([lv, rv], axis=-1).reshape(
        1, 1, 1, _KSUM, _C)


_assemble_call = pl.pallas_call(
    _assemble_body,
    grid=(_NBT,),
    in_specs=[
        pl.BlockSpec((1, _KPAD_SUM, 128), lambda i: (i, 0, 0)),
        pl.BlockSpec((1, _KPAD_SUM, 128), lambda i: (i, 0, 0)),
    ],
    out_specs=pl.BlockSpec(
        (1, 1, 1, _KSUM, _C),
        lambda i: (i // (_N * _T), (i // _T) % _N, i % _T, 0, 0)),
    out_shape=jax.ShapeDtypeStruct((_B, _N, _T, _KSUM, _C), jnp.float32),
)


def kernel(src0, src1, src2, src3, scan_queries, ln_w, ln_b):
    srcs = (src0, src1, src2, src3)
    w2 = ln_w.reshape(1, _C)
    b2 = ln_b.reshape(1, _C)
    tables = []
    for s, src in enumerate(srcs):
        hw = _HWS[s]
        tables.append(jnp.transpose(src.reshape(_B, _C, _T, hw),
                                    (0, 2, 3, 1)).reshape(_B * _T * hw, _C))
    idxs = [None] * 4
    for s in (3, 2, 1, 0):
        hw = _HWS[s]
        (idx,) = _make_scale_call(hw, _KS[s], _KPADS[s])(
            tables[s], scan_queries, w2, b2)
        idxs[s] = idx.reshape(-1)
    outl, outr = _sc_gather(tables, idxs)
    return _assemble_call(outl.reshape(_NBT, _KPAD_SUM, 128),
                          outr.reshape(_NBT, _KPAD_SUM, 128))


# speedup vs baseline: 1.2852x; 1.1665x over previous
import functools
import math

import jax
import jax.numpy as jnp
import numpy as np
from jax import lax
from jax.experimental import pallas as pl
from jax.experimental.pallas import tpu as pltpu
from jax.experimental.pallas import tpu_sc as plsc

_B = 2
_N = 20
_C = 256
_T = 4
_HWS = (64 * 64, 32 * 32, 16 * 16, 8 * 8)
_KS = tuple(int(math.floor(0.0625 * hw)) for hw in _HWS)
_KPADS = tuple(max(8, k) for k in _KS)
_KOFFS = (0, 256, 320, 336)
_KSUM = sum(_KS)
_KPAD_SUM = 344
_NBT = _B * _N * _T

_MSB = np.int32(-2147483648)
_LOW31 = np.int32(2147483647)


def _monotone_key(x_f32):
    fi = lax.bitcast_convert_type(x_f32, jnp.int32)
    return jnp.where(fi >= 0, fi, fi ^ _LOW31)


def _row_cumsum(x, hw):
    n = x.shape[0]
    if hw <= 128:
        io_r = lax.broadcasted_iota(jnp.int32, (hw, hw), 0)
        io_c = lax.broadcasted_iota(jnp.int32, (hw, hw), 1)
        tri_incl = (io_r <= io_c).astype(jnp.float32)
        return jnp.dot(x, tri_incl, preferred_element_type=jnp.float32)
    r = hw // 128
    io_r = lax.broadcasted_iota(jnp.int32, (128, 128), 0)
    io_c = lax.broadcasted_iota(jnp.int32, (128, 128), 1)
    tri_incl = (io_r <= io_c).astype(jnp.float32)
    within = jnp.dot(x.reshape(n * r, 128), tri_incl,
                     preferred_element_type=jnp.float32).reshape(n, r, 128)
    blk_sum = jnp.sum(x.reshape(n, r, 128), axis=2)
    io_br = lax.broadcasted_iota(jnp.int32, (r, r), 0)
    io_bc = lax.broadcasted_iota(jnp.int32, (r, r), 1)
    tri_excl = (io_br < io_bc).astype(jnp.float32)
    base = jnp.dot(blk_sum, tri_excl, preferred_element_type=jnp.float32)
    return (within + base[:, :, None]).reshape(n, hw)


def _topk_idx(mask, hw, k):
    n = mask.shape[0]
    ikey = _monotone_key(mask)
    tpat = jnp.zeros((n, 1), jnp.int32)
    for bit in range(31, -1, -1):
        bitc = np.int32(np.uint32(1 << bit))
        trial = tpat | bitc
        s_trial = trial ^ _MSB
        cnt = jnp.sum((ikey >= s_trial).astype(jnp.float32), axis=1,
                      keepdims=True)
        tpat = jnp.where(cnt >= float(k), trial, tpat)
    s_thr = tpat ^ _MSB
    gt = ikey > s_thr
    eq = ikey == s_thr
    cnt_gt = jnp.sum(gt.astype(jnp.float32), axis=1, keepdims=True)
    m_tie = float(k) - cnt_gt
    eqcum = _row_cumsum(eq.astype(jnp.float32), hw)
    sel = jnp.logical_or(gt, jnp.logical_and(eq, eqcum <= m_tie))
    selcum = _row_cumsum(sel.astype(jnp.float32), hw)

    chunks = []
    nch = (k + 7) // 8
    for c0 in range(nch):
        jv = lax.broadcasted_iota(jnp.int32, (1, 8, 1), 1).astype(
            jnp.float32) + float(c0 * 8)
        cmp = (selcum[:, None, :] <= jv).astype(jnp.float32)
        chunks.append(jnp.sum(cmp, axis=2))
    pos = jnp.concatenate(chunks, axis=1)[:, :k]
    return pos.astype(jnp.int32)


def _layer_norm_q(q_ref, w_ref, b_ref):
    q = q_ref[...].reshape(_N, _C)
    mu = jnp.mean(q, axis=-1, keepdims=True)
    var = jnp.mean((q - mu) ** 2, axis=-1, keepdims=True)
    return (q - mu) / jnp.sqrt(var + 1e-5) * w_ref[...].reshape(1, _C) \
        + b_ref[...].reshape(1, _C)


def _frame_body(hw, k, k_pad, srct_ref, q_ref, w_ref, b_ref, idx_ref):
    i = pl.program_id(0)
    qn = _layer_norm_q(q_ref, w_ref, b_ref)
    srct = srct_ref[...]
    maskt = jnp.dot(srct, qn.T, preferred_element_type=jnp.float32)
    mask = maskt.T
    idx = _topk_idx(mask, hw, k) + i * hw
    if k_pad > k:
        idx = jnp.concatenate(
            [idx, jnp.zeros((_N, k_pad - k), jnp.int32)], axis=1)
    idx_ref[...] = idx.reshape(1, 1, _N, k_pad)


def _make_scale_call(hw, k, k_pad):
    body = functools.partial(_frame_body, hw, k, k_pad)
    return pl.pallas_call(
        body,
        grid=(_B * _T,),
        in_specs=[
            pl.BlockSpec((hw, _C), lambda i: (i, 0)),
            pl.BlockSpec((1, _N, _C), lambda i: (i // _T, 0, 0)),
            pl.BlockSpec((1, _C), lambda i: (0, 0)),
            pl.BlockSpec((1, _C), lambda i: (0, 0)),
        ],
        out_specs=[
            pl.BlockSpec((1, 1, _N, k_pad), lambda i: (i // _T, i % _T, 0, 0)),
        ],
        out_shape=[
            jax.ShapeDtypeStruct((_B, _T, _N, k_pad), jnp.int32),
        ],
    )


def _sc_gather(tables, idxs):
    mesh = plsc.VectorSubcoreMesh(core_axis_name="c", subcore_axis_name="s")
    rows_per_tile = _NBT // 32

    @functools.partial(
        pl.kernel,
        mesh=mesh,
        out_type=[
            jax.ShapeDtypeStruct((_NBT * _KPAD_SUM, 128), jnp.float32),
            jax.ShapeDtypeStruct((_NBT * _KPAD_SUM, 128), jnp.float32),
        ],
        scratch_types=[
            pltpu.VMEM((128,), jnp.int32),
            pltpu.VMEM((128, _C), jnp.float32),
            pltpu.SemaphoreType.DMA,
        ],
    )
    def k(t0h, t1h, t2h, t3h, i0h, i1h, i2h, i3h, outl, outr,
          idx_v, rows_v, sem):
        wid = lax.axis_index("s") * 2 + lax.axis_index("c")
        tbls = (t0h, t1h, t2h, t3h)
        idxs_h = (i0h, i1h, i2h, i3h)

        def task(a, carry):
            r = wid * rows_per_tile + a
            b = r // (_N * _T)
            n = (r // _T) % _N
            t = r % _T
            bt = b * _T + t
            for s in range(4):
                kk, kp, off = _KS[s], _KPADS[s], _KOFFS[s]
                idx_off = (bt * _N + n) * kp
                for c0 in range(0, kp, 128):
                    cw = min(128, kp - c0)
                    wr = min(kk - c0, cw)
                    idx_sub = idx_v.at[pl.ds(0, cw)]
                    pltpu.sync_copy(idxs_h[s].at[pl.ds(idx_off + c0, cw)],
                                    idx_sub)
                    pltpu.async_copy(tbls[s].at[idx_sub],
                                     rows_v.at[pl.ds(0, cw)], sem).wait()
                    qbase = r * _KPAD_SUM + off + c0
                    pltpu.sync_copy(rows_v.at[pl.ds(0, wr), pl.ds(0, 128)],
                                    outl.at[pl.ds(qbase, wr), :])
                    pltpu.sync_copy(rows_v.at[pl.ds(0, wr), pl.ds(128, 128)],
                                    outr.at[pl.ds(qbase, wr), :])
            return carry

        lax.fori_loop(0, rows_per_tile, task, 0)

    return k(*tables, *idxs)


_NCH = 5


def _assemble_body(l_ref, r_ref, out_ref):
    lv = l_ref[...].reshape(_NCH, _T, _KPAD_SUM, 128)[:, :, :_KSUM, :]
    rv = r_ref[...].reshape(_NCH, _T, _KPAD_SUM, 128)[:, :, :_KSUM, :]
    out_ref[...] = jnp.concatenate([lv, rv], axis=-1).reshape(
        1, _NCH, _T, _KSUM, _C)


_assemble_call = pl.pallas_call(
    _assemble_body,
    grid=(_NBT // (_NCH * _T),),
    in_specs=[
        pl.BlockSpec((_NCH * _T, _KPAD_SUM, 128), lambda i: (i, 0, 0)),
        pl.BlockSpec((_NCH * _T, _KPAD_SUM, 128), lambda i: (i, 0, 0)),
    ],
    out_specs=pl.BlockSpec(
        (1, _NCH, _T, _KSUM, _C),
        lambda i: (i // (_N // _NCH), i % (_N // _NCH), 0, 0, 0)),
    out_shape=jax.ShapeDtypeStruct((_B, _N, _T, _KSUM, _C), jnp.float32),
)


def kernel(src0, src1, src2, src3, scan_queries, ln_w, ln_b):
    srcs = (src0, src1, src2, src3)
    w2 = ln_w.reshape(1, _C)
    b2 = ln_b.reshape(1, _C)
    tables = []
    for s, src in enumerate(srcs):
        hw = _HWS[s]
        tables.append(jnp.transpose(src.reshape(_B, _C, _T, hw),
                                    (0, 2, 3, 1)).reshape(_B * _T * hw, _C))
    idxs = [None] * 4
    for s in (3, 2, 1, 0):
        hw = _HWS[s]
        (idx,) = _make_scale_call(hw, _KS[s], _KPADS[s])(
            tables[s], scan_queries, w2, b2)
        idxs[s] = idx.reshape(-1)
    outl, outr = _sc_gather(tables, idxs)
    return _assemble_call(outl.reshape(_NBT, _KPAD_SUM, 128),
                          outr.reshape(_NBT, _KPAD_SUM, 128))
